# baseline (speedup 1.0000x reference)
import functools

import jax
import jax.numpy as jnp
from jax import lax
from jax.experimental import pallas as pl
from jax.experimental.pallas import tpu as pltpu
from jax.experimental.pallas import tpu_sc as plsc

VOCAB = 100000
EMB = 300
B = 1024
L = 200
HID = 256
TGT = 20

CHUNKA = 128
CHUNKB = L - CHUNKA
LP = 256
NW = 32
ROWS_PER_W = B // NW
IDXROWS_PER_W = ROWS_PER_W * 2
CBASE2 = 256

_CHUNKS = (
    tuple((16 * k, k) for k in range(8)),
    tuple((16 * k, 8 + k) for k in range(8)),
    ((0, 16), (16, 17), (284 - CBASE2, 18)),
)
_NACC = 19


def _accumulate(buf, chunks, n_rows, accs):
    def body(r, accs):
        sums, maxs = accs
        sums, maxs = list(sums), list(maxs)
        for off, ai in chunks:
            v = buf[r, pl.ds(off, 16)]
            sums[ai] = sums[ai] + v
            maxs[ai] = jnp.maximum(maxs[ai], v)
        return (tuple(sums), tuple(maxs))

    return lax.fori_loop(0, n_rows, body, accs, unroll=4)


def _sc_body(x1_hbm, t_hbm, out_hbm, idx_v, *rest):
    bufs = rest[:6]
    stage = rest[6]
    sems = rest[7:13]
    cid = lax.axis_index("c")
    sid = lax.axis_index("s")
    w = sid * 2 + cid

    def src(c, pos, b):
        if pos == 0:
            idx = idx_v.at[2 * b]
        else:
            idx = idx_v.at[2 * b + 1, pl.ds(0, CHUNKB)]
        return t_hbm.at[c].at[idx]

    pltpu.sync_copy(x1_hbm.at[pl.ds(w * IDXROWS_PER_W, IDXROWS_PER_W)],
                    idx_v)

    for pos in range(2):
        for c in range(3):
            k = pos * 3 + c
            pltpu.async_copy(src(c, pos, 0), bufs[k], sems[k])

    inv_l = jnp.float32(1.0 / L)

    def row_body(b, carry):
        accs = (
            tuple(jnp.zeros((16,), jnp.float32) for _ in range(_NACC)),
            tuple(jnp.full((16,), -jnp.inf, jnp.float32) for _ in range(_NACC)),
        )
        for pos in range(2):
            n_rows = CHUNKA if pos == 0 else CHUNKB
            for c in range(3):
                k = pos * 3 + c
                pltpu.make_async_copy(src(c, pos, b), bufs[k], sems[k]).wait()
                accs = _accumulate(bufs[k], _CHUNKS[c], n_rows, accs)
                nxt = jnp.minimum(b + 1, ROWS_PER_W - 1)
                pltpu.async_copy(src(c, pos, nxt), bufs[k], sems[k])

        sums, maxs = accs
        stage[pl.ds(EMB - 16, 16)] = sums[18] * inv_l
        stage[pl.ds(2 * EMB - 16, 16)] = maxs[18]
        for i in range(18):
            stage[pl.ds(16 * i, 16)] = sums[i] * inv_l
            stage[pl.ds(EMB + 16 * i, 16)] = maxs[i]
        pltpu.sync_copy(stage, out_hbm.at[w * ROWS_PER_W + b])
        return carry

    lax.fori_loop(0, ROWS_PER_W, row_body, None)

    for pos in range(2):
        for c in range(3):
            k = pos * 3 + c
            pltpu.make_async_copy(src(c, pos, ROWS_PER_W - 1), bufs[k],
                                  sems[k]).wait()


_sc_pool = functools.partial(
    pl.kernel,
    out_type=jax.ShapeDtypeStruct((B, 2 * EMB), jnp.float32),
    mesh=plsc.VectorSubcoreMesh(core_axis_name="c", subcore_axis_name="s"),
    compiler_params=pltpu.CompilerParams(use_tc_tiling_on_sc=False),
    scratch_types=(
        [pltpu.VMEM((IDXROWS_PER_W, 128), jnp.int32)]
        + [pltpu.VMEM((CHUNKA, 128), jnp.float32) for _ in range(3)]
        + [pltpu.VMEM((CHUNKB, 128), jnp.float32) for _ in range(3)]
        + [pltpu.VMEM((2 * EMB,), jnp.float32)]
        + [pltpu.SemaphoreType.DMA for _ in range(6)]
    ),
)(_sc_body)


def _mlp_body(h_ref, g1_ref, b1_ref, w1t_ref, bias1_ref, g2_ref, b2_ref,
              w2t_ref, bias2_ref, out_ref, hid_ref):
    h = h_ref[...]
    mu = jnp.mean(h, axis=0, keepdims=True)
    d = h - mu
    var = jnp.mean(d * d, axis=0, keepdims=True)
    hn = d * lax.rsqrt(var + 1e-5) * g1_ref[...] + b1_ref[...]
    h1 = jnp.dot(hn, w1t_ref[...], preferred_element_type=jnp.float32,
                 precision=lax.Precision.HIGHEST) + bias1_ref[...]
    hid_ref[...] = h1
    mu2 = jnp.mean(h1, axis=0, keepdims=True)
    d2 = h1 - mu2
    var2 = jnp.mean(d2 * d2, axis=0, keepdims=True)
    h2 = d2 * lax.rsqrt(var2 + 1e-5) * g2_ref[...] + b2_ref[...]
    out_ref[...] = jnp.dot(h2, w2t_ref[...], preferred_element_type=jnp.float32,
                           precision=lax.Precision.HIGHEST) + bias2_ref[...]


_mlp = pl.pallas_call(
    _mlp_body,
    out_shape=(
        jax.ShapeDtypeStruct((B, TGT), jnp.float32),
        jax.ShapeDtypeStruct((B, HID), jnp.float32),
    ),
)


def kernel(x, emb, g1, b1, W1, bias1, g2, b2, W2, bias2):
    x1 = jnp.pad(x, ((0, 0), (0, LP - L))).reshape(B * 2, 128)
    p2 = jnp.pad(emb[:, CBASE2:EMB], ((0, 0), (0, 128 - (EMB - CBASE2))))
    t = jnp.stack([emb[:, 0:128], emb[:, 128:256], p2], axis=0)
    h = _sc_pool(x1, t)
    out, hid = _mlp(h, g1.reshape(1, -1), b1.reshape(1, -1), W1.T,
                    bias1.reshape(1, -1), g2.reshape(1, -1),
                    b2.reshape(1, -1), W2.T, bias2.reshape(1, -1))
    return (out, hid)
